# baseline (speedup 1.0000x reference)
import functools

import jax
import jax.numpy as jnp
from jax import lax
from jax.experimental import pallas as pl
from jax.experimental.pallas import tpu as pltpu
from jax.experimental.pallas import tpu_sc as plsc

_HID = 64
_LQ = 2048
_LK = 2048
_BIG = 4096
_NW = 32
_RPW = _LQ // _NW


def _build_body(off_ref, w_ref, big_ref):
    off = off_ref[0]
    big_ref[:, :] = jnp.broadcast_to(w_ref[0:1, :], (_BIG, _HID))
    big_ref[pl.ds(1536 - off, 1023), :] = w_ref[pl.ds(0, 1023), :]
    big_ref[pl.ds(2559 - off, 1537), :] = jnp.broadcast_to(
        w_ref[1022:1023, :], (1537, _HID)
    )


_mesh = plsc.VectorSubcoreMesh(core_axis_name="c", subcore_axis_name="s")


@functools.partial(
    pl.kernel,
    out_type=jax.ShapeDtypeStruct((_LQ, _LK, _HID), jnp.float32),
    mesh=_mesh,
    scratch_types=[pltpu.MemorySpace.VMEM_SHARED((_BIG, _HID), jnp.float32)],
)
def _sc_copy(big_hbm, out_hbm, big_sh):
    c = lax.axis_index("c")
    s = lax.axis_index("s")

    @pl.when(s == 0)
    def _stage():
        pltpu.sync_copy(big_hbm, big_sh)

    plsc.subcore_barrier()

    wid = s * 2 + c

    def one(n, carry):
        i = wid * _RPW + n
        start = _LK - 1 - i
        pltpu.sync_copy(big_sh.at[pl.ds(start, _LK), :], out_hbm.at[i])
        return carry

    lax.fori_loop(0, _RPW, one, 0)


def kernel(weight, length_q, length_k):
    off = jnp.asarray(
        (length_q - _LQ) + (length_k - _LK), dtype=jnp.int32
    ).reshape((1,))
    big = pl.pallas_call(
        _build_body,
        in_specs=[
            pl.BlockSpec(memory_space=pltpu.MemorySpace.SMEM),
            pl.BlockSpec(memory_space=pltpu.MemorySpace.VMEM),
        ],
        out_specs=pl.BlockSpec(memory_space=pltpu.MemorySpace.VMEM),
        out_shape=jax.ShapeDtypeStruct((_BIG, _HID), jnp.float32),
    )(off, weight)
    return _sc_copy(big)


# speedup vs baseline: 1.0035x; 1.0035x over previous
import functools

import jax
import jax.numpy as jnp
from jax import lax
from jax.experimental import pallas as pl
from jax.experimental.pallas import tpu as pltpu
from jax.experimental.pallas import tpu_sc as plsc

_HID = 64
_LQ = 2048
_LK = 2048
_BIG = 4096
_NW = 32
_RPW = _LQ // _NW


def _build_body(off_ref, w_ref, big_ref):
    off = off_ref[0]
    big_ref[:, :] = jnp.broadcast_to(w_ref[0:1, :], (_BIG, _HID))
    big_ref[pl.ds(1536 - off, 1023), :] = w_ref[pl.ds(0, 1023), :]
    big_ref[pl.ds(2559 - off, 1537), :] = jnp.broadcast_to(
        w_ref[1022:1023, :], (1537, _HID)
    )


_mesh = plsc.VectorSubcoreMesh(core_axis_name="c", subcore_axis_name="s")


_RING = 4


@functools.partial(
    pl.kernel,
    out_type=jax.ShapeDtypeStruct((_LQ, _LK, _HID), jnp.float32),
    mesh=_mesh,
    scratch_types=[
        pltpu.MemorySpace.VMEM_SHARED((_BIG, _HID), jnp.float32),
        [pltpu.SemaphoreType.DMA] * _RING,
    ],
)
def _sc_copy(big_hbm, out_hbm, big_sh, sems):
    c = lax.axis_index("c")
    s = lax.axis_index("s")

    @pl.when(s == 0)
    def _stage():
        pltpu.sync_copy(big_hbm, big_sh)

    plsc.subcore_barrier()

    wid = s * 2 + c
    base = wid * _RPW

    def issue(n, slot):
        start = _LK - 1 - (base + n)
        pltpu.make_async_copy(
            big_sh.at[pl.ds(start, _LK), :], out_hbm.at[base + n], sems[slot]
        ).start()

    def drain(n, slot):
        pltpu.make_async_copy(
            big_sh.at[pl.ds(0, _LK), :], out_hbm.at[base + n], sems[slot]
        ).wait()

    for k in range(_RING):
        issue(k, k)

    def chunk(cc, carry):
        b = cc * _RING
        for k in range(_RING):
            drain(b + k - _RING, k)
            issue(b + k, k)
        return carry

    lax.fori_loop(1, _RPW // _RING, chunk, 0)

    for k in range(_RING):
        drain(_RPW - _RING + k, k)


def kernel(weight, length_q, length_k):
    off = jnp.asarray(
        (length_q - _LQ) + (length_k - _LK), dtype=jnp.int32
    ).reshape((1,))
    big = pl.pallas_call(
        _build_body,
        in_specs=[
            pl.BlockSpec(memory_space=pltpu.MemorySpace.SMEM),
            pl.BlockSpec(memory_space=pltpu.MemorySpace.VMEM),
        ],
        out_specs=pl.BlockSpec(memory_space=pltpu.MemorySpace.VMEM),
        out_shape=jax.ShapeDtypeStruct((_BIG, _HID), jnp.float32),
    )(off, weight)
    return _sc_copy(big)


# speedup vs baseline: 4.5462x; 4.5304x over previous
import jax
import jax.numpy as jnp
from jax.experimental import pallas as pl
from jax.experimental.pallas import tpu as pltpu

_HID = 64
_LQ = 2048
_LK = 2048
_BIGW = 4224
_BI = 4


def _build_body(off_ref, w_ref, big_ref):
    off = off_ref[0]
    big_ref[:, :] = jnp.broadcast_to(w_ref[0:1, :], (_BIGW, _HID))
    big_ref[pl.ds(1536 - off, 1023), :] = w_ref[pl.ds(0, 1023), :]
    big_ref[pl.ds(2559 - off, 1665), :] = jnp.broadcast_to(
        w_ref[1022:1023, :], (1665, _HID)
    )


def _main_body(off_ref, bigt_ref, out_ref):
    g = pl.program_id(0)
    off = off_ref[0]
    for r in range(_BI):
        i = g * _BI + r
        s = jnp.clip(_LK - 1 - i + off, 0, 2048)
        sa = pl.multiple_of((s // 128) * 128, 128)
        p = s - sa
        win = bigt_ref[:, pl.ds(sa, _LK + 128)]
        out_ref[r] = pltpu.roll(win, -p, axis=1)[:, :_LK]


def kernel(weight, length_q, length_k):
    off = jnp.asarray(
        (length_q - _LQ) + (length_k - _LK), dtype=jnp.int32
    ).reshape((1,))
    big = pl.pallas_call(
        _build_body,
        in_specs=[
            pl.BlockSpec(memory_space=pltpu.MemorySpace.SMEM),
            pl.BlockSpec(memory_space=pltpu.MemorySpace.VMEM),
        ],
        out_specs=pl.BlockSpec(memory_space=pltpu.MemorySpace.VMEM),
        out_shape=jax.ShapeDtypeStruct((_BIGW, _HID), jnp.float32),
    )(off, weight)
    bigt = jnp.transpose(big)
    out_t = pl.pallas_call(
        _main_body,
        grid=(_LQ // _BI,),
        in_specs=[
            pl.BlockSpec(memory_space=pltpu.MemorySpace.SMEM),
            pl.BlockSpec((_HID, _BIGW), lambda g: (0, 0)),
        ],
        out_specs=pl.BlockSpec((_BI, _HID, _LK), lambda g: (g, 0, 0)),
        out_shape=jax.ShapeDtypeStruct((_LQ, _HID, _LK), jnp.float32),
    )(off, bigt)
    return jnp.transpose(out_t, (0, 2, 1))
